# Initial kernel scaffold; baseline (speedup 1.0000x reference)
import jax
import jax.numpy as jnp
from jax.experimental import pallas as pl


def kernel(x, edge_index, W_embed, b_embed, conv_W, conv_b, bn_gamma, bn_beta, bn_mean, bn_var):
    raise NotImplementedError("write your pallas kernel here")


# speedup vs baseline: 12.7232x; 12.7232x over previous
import dataclasses
import functools

import jax
import jax.numpy as jnp
from jax import lax
from jax.experimental import pallas as pl
from jax.experimental.pallas import tpu as pltpu
from jax.experimental.pallas import tpu_sc as plsc

_NC = 2
_NS = 16
_NW = _NC * _NS
_K = 128
_H = 64
_RB = 1000


def _ceil_to(a, m):
    return (a + m - 1) // m * m


def _sc_mesh():
    return plsc.VectorSubcoreMesh(core_axis_name="c", subcore_axis_name="s")


def _sc_params():
    return pltpu.CompilerParams(use_tc_tiling_on_sc=False)


def _sc_degree(dst_pad, acc_rows, nb):
    rows_per_sub = acc_rows // _NS

    @functools.partial(
        pl.kernel,
        out_type=jax.ShapeDtypeStruct((_NC, acc_rows, 16), jnp.float32),
        mesh=_sc_mesh(),
        compiler_params=_sc_params(),
        scratch_types=[
            pltpu.VMEM((_K,), jnp.int32),
            pltpu.VMEM((_K, 16), jnp.float32),
            pltpu.VMEM((128, 16), jnp.float32),
            pltpu.VMEM_SHARED((acc_rows, 16), jnp.float32),
        ],
    )
    def deg_kernel(dst_hbm, out_hbm, didx, ones_v, zero_v, acc):
        c = lax.axis_index("c")
        s = lax.axis_index("s")

        @pl.loop(0, _K)
        def _(r):
            ones_v[r, pl.ds(0, 16)] = jnp.full((16,), 1.0, jnp.float32)

        @pl.loop(0, 128)
        def _(r):
            zero_v[r, pl.ds(0, 16)] = jnp.zeros((16,), jnp.float32)

        base_rows = s * rows_per_sub

        @pl.loop(0, rows_per_sub // 128)
        def _(b):
            pltpu.sync_copy(zero_v, acc.at[pl.ds(base_rows + b * 128, 128)])

        plsc.subcore_barrier()

        wid = s * _NC + c
        base_e = wid * (nb * _K)

        @pl.loop(0, nb)
        def _(b):
            pltpu.sync_copy(dst_hbm.at[pl.ds(base_e + b * _K, _K)], didx)
            pltpu.sync_copy(ones_v, acc.at[didx], add=True)

        plsc.subcore_barrier()
        pltpu.sync_copy(
            acc.at[pl.ds(base_rows, rows_per_sub)],
            out_hbm.at[c].at[pl.ds(base_rows, rows_per_sub)],
        )

    return deg_kernel(dst_pad)


def _sc_edge_pass(p, src_pad, dst_pad, acc_rows, nb):
    rows_per_sub = acc_rows // _NS

    @functools.partial(
        pl.kernel,
        out_type=jax.ShapeDtypeStruct((_NC, acc_rows, _H), jnp.float32),
        mesh=_sc_mesh(),
        compiler_params=_sc_params(),
        scratch_types=[
            pltpu.VMEM((_K,), jnp.int32),
            pltpu.VMEM((_K,), jnp.int32),
            pltpu.VMEM((_K, _H), jnp.float32),
            pltpu.VMEM((128, _H), jnp.float32),
            pltpu.VMEM_SHARED((acc_rows, _H), jnp.float32),
        ],
    )
    def edge_kernel(p_hbm, src_hbm, dst_hbm, out_hbm, sidx, didx, rows, zero_v, acc):
        c = lax.axis_index("c")
        s = lax.axis_index("s")

        @pl.loop(0, 128)
        def _(r):
            @pl.loop(0, _H, step=16)
            def _(j):
                zero_v[r, pl.ds(j, 16)] = jnp.zeros((16,), jnp.float32)

        base_rows = s * rows_per_sub

        @pl.loop(0, rows_per_sub // 128)
        def _(b):
            pltpu.sync_copy(zero_v, acc.at[pl.ds(base_rows + b * 128, 128)])

        plsc.subcore_barrier()

        wid = s * _NC + c
        base_e = wid * (nb * _K)

        @pl.loop(0, nb)
        def _(b):
            off = base_e + b * _K
            pltpu.sync_copy(src_hbm.at[pl.ds(off, _K)], sidx)
            pltpu.sync_copy(dst_hbm.at[pl.ds(off, _K)], didx)
            pltpu.sync_copy(p_hbm.at[sidx], rows)
            pltpu.sync_copy(rows, acc.at[didx], add=True)

        plsc.subcore_barrier()
        pltpu.sync_copy(
            acc.at[pl.ds(base_rows, rows_per_sub)],
            out_hbm.at[c].at[pl.ds(base_rows, rows_per_sub)],
        )

    return edge_kernel(p, src_pad, dst_pad)


def _embed_body(x_ref, w_ref, b_ref, o_ref):
    h = jnp.dot(x_ref[...], w_ref[...], preferred_element_type=jnp.float32)
    o_ref[...] = jnp.maximum(h + b_ref[...], 0.0)


def _p0_body(deg_ref, h_ref, w_ref, dinv_ref, p_ref):
    deg = 1.0 + deg_ref[0, :, 0] + deg_ref[1, :, 0]
    dinv = (1.0 / jnp.sqrt(deg))[:, None]
    dinv_ref[...] = dinv
    p_ref[...] = dinv * jnp.dot(h_ref[...], w_ref[...],
                                preferred_element_type=jnp.float32)


def _combine_body(part_ref, p_ref, h_ref, dinv_ref, cb_ref, g_ref, be_ref,
                  mu_ref, var_ref, wn_ref, hn_ref, pn_ref):
    dinv = dinv_ref[...]
    agg = part_ref[0] + part_ref[1] + p_ref[...]
    t = dinv * agg + cb_ref[...]
    inv = 1.0 / jnp.sqrt(var_ref[...] + 1e-5)
    t = (t - mu_ref[...]) * inv * g_ref[...] + be_ref[...]
    hn = jnp.maximum(t, 0.0) + h_ref[...]
    hn_ref[...] = hn
    pn_ref[...] = dinv * jnp.dot(hn, wn_ref[...],
                                 preferred_element_type=jnp.float32)


def kernel(x, edge_index, W_embed, b_embed, conv_W, conv_b, bn_gamma, bn_beta,
           bn_mean, bn_var):
    N, F_in = x.shape
    E = edge_index.shape[1]
    L = conv_W.shape[0]

    acc_rows = _ceil_to(N + 1, _NS * 128)
    padded_e = _ceil_to(E, _NW * _K)
    nb = padded_e // (_NW * _K)
    pad = padded_e - E

    src_pad = jnp.concatenate(
        [edge_index[0], jnp.zeros((pad,), edge_index.dtype)])
    dst_pad = jnp.concatenate(
        [edge_index[1], jnp.full((pad,), N, edge_index.dtype)])

    grid = (N // _RB,)
    row_spec = pl.BlockSpec((_RB, _H), lambda i: (i, 0))
    vec_spec = pl.BlockSpec((1, _H), lambda i: (0, 0))
    part_spec = pl.BlockSpec((_NC, _RB, _H), lambda i: (0, i, 0))
    f32 = jnp.float32

    deg_part = _sc_degree(dst_pad, acc_rows, nb)

    h = pl.pallas_call(
        _embed_body,
        grid=grid,
        in_specs=[pl.BlockSpec((_RB, F_in), lambda i: (i, 0)),
                  pl.BlockSpec((F_in, _H), lambda i: (0, 0)),
                  vec_spec],
        out_specs=row_spec,
        out_shape=jax.ShapeDtypeStruct((N, _H), f32),
    )(x, W_embed, b_embed.reshape(1, _H))

    dinv, p = pl.pallas_call(
        _p0_body,
        grid=grid,
        in_specs=[pl.BlockSpec((_NC, _RB, 16), lambda i: (0, i, 0)),
                  row_spec,
                  pl.BlockSpec((_H, _H), lambda i: (0, 0))],
        out_specs=[pl.BlockSpec((_RB, 1), lambda i: (i, 0)), row_spec],
        out_shape=[jax.ShapeDtypeStruct((N, 1), f32),
                   jax.ShapeDtypeStruct((N, _H), f32)],
    )(deg_part, h, conv_W[0])

    combine = pl.pallas_call(
        _combine_body,
        grid=grid,
        in_specs=[part_spec, row_spec, row_spec,
                  pl.BlockSpec((_RB, 1), lambda i: (i, 0)),
                  vec_spec, vec_spec, vec_spec, vec_spec, vec_spec,
                  pl.BlockSpec((_H, _H), lambda i: (0, 0))],
        out_specs=[row_spec, row_spec],
        out_shape=[jax.ShapeDtypeStruct((N, _H), f32),
                   jax.ShapeDtypeStruct((N, _H), f32)],
    )

    g2 = bn_gamma.reshape(1, _H)
    be2 = bn_beta.reshape(1, _H)
    mu2 = bn_mean.reshape(1, _H)
    var2 = bn_var.reshape(1, _H)

    for i in range(L):
        part = _sc_edge_pass(p, src_pad, dst_pad, acc_rows, nb)
        w_next = conv_W[(i + 1) % L]
        h, p = combine(part, p, h, dinv, conv_b[i].reshape(1, _H),
                       g2, be2, mu2, var2, w_next)
    return h


# speedup vs baseline: 13.5158x; 1.0623x over previous
import dataclasses
import functools

import jax
import jax.numpy as jnp
from jax import lax
from jax.experimental import pallas as pl
from jax.experimental.pallas import tpu as pltpu
from jax.experimental.pallas import tpu_sc as plsc

_NC = 2
_NS = 16
_NW = _NC * _NS
_K = 128
_H = 64
_RB = 1000


def _ceil_to(a, m):
    return (a + m - 1) // m * m


def _sc_mesh():
    return plsc.VectorSubcoreMesh(core_axis_name="c", subcore_axis_name="s")


def _sc_params():
    return pltpu.CompilerParams(use_tc_tiling_on_sc=False)


def _sc_degree(dst_t, acc_rows, nb):
    rows_per_sub = acc_rows // _NS
    chunk = 8

    @functools.partial(
        pl.kernel,
        out_type=jax.ShapeDtypeStruct((_NC, acc_rows, 16), jnp.float32),
        mesh=_sc_mesh(),
        compiler_params=_sc_params(),
        scratch_types=[
            pltpu.VMEM((nb, _K), jnp.int32),
            pltpu.VMEM((_K, 16), jnp.float32),
            pltpu.VMEM((128, 16), jnp.float32),
            pltpu.SemaphoreType.DMA,
            pltpu.VMEM_SHARED((acc_rows, 16), jnp.float32),
        ],
    )
    def deg_kernel(dst_hbm, out_hbm, didx, ones_v, zero_v, sem, acc):
        c = lax.axis_index("c")
        s = lax.axis_index("s")
        wid = s * _NC + c

        pltpu.sync_copy(dst_hbm.at[wid], didx)

        @pl.loop(0, _K)
        def _(r):
            ones_v[r, pl.ds(0, 16)] = jnp.full((16,), 1.0, jnp.float32)

        @pl.loop(0, 128)
        def _(r):
            zero_v[r, pl.ds(0, 16)] = jnp.zeros((16,), jnp.float32)

        base_rows = s * rows_per_sub

        @pl.loop(0, rows_per_sub // 128)
        def _(b):
            pltpu.sync_copy(zero_v, acc.at[pl.ds(base_rows + b * 128, 128)])

        plsc.subcore_barrier()

        @pl.loop(0, nb, step=chunk)
        def _(b):
            @pl.loop(0, chunk)
            def _(j):
                pltpu.async_copy(ones_v, acc.at[didx.at[b + j]], sem,
                                 add=True)

            @pl.loop(0, chunk)
            def _(j):
                pltpu.make_async_copy(ones_v, acc.at[didx.at[b + j]],
                                      sem).wait()

        plsc.subcore_barrier()
        pltpu.sync_copy(
            acc.at[pl.ds(base_rows, rows_per_sub)],
            out_hbm.at[c].at[pl.ds(base_rows, rows_per_sub)],
        )

    return deg_kernel(dst_t)


_NBUF = 4


def _sc_edge_pass(p, src_t, dst_t, acc_rows, nb):
    rows_per_sub = acc_rows // _NS

    @functools.partial(
        pl.kernel,
        out_type=jax.ShapeDtypeStruct((_NC, acc_rows, _H), jnp.float32),
        mesh=_sc_mesh(),
        compiler_params=_sc_params(),
        scratch_types=[
            pltpu.VMEM((nb, _K), jnp.int32),
            pltpu.VMEM((nb, _K), jnp.int32),
            [pltpu.VMEM((_K, _H), jnp.float32) for _ in range(_NBUF)],
            [pltpu.SemaphoreType.DMA for _ in range(_NBUF)],
            pltpu.VMEM((128, _H), jnp.float32),
            pltpu.VMEM_SHARED((acc_rows, _H), jnp.float32),
        ],
    )
    def edge_kernel(p_hbm, src_hbm, dst_hbm, out_hbm, sidx, didx, rows, gsem,
                    zero_v, acc):
        c = lax.axis_index("c")
        s = lax.axis_index("s")
        wid = s * _NC + c

        pltpu.sync_copy(src_hbm.at[wid], sidx)
        pltpu.sync_copy(dst_hbm.at[wid], didx)

        @pl.loop(0, 128)
        def _(r):
            @pl.loop(0, _H, step=16)
            def _(j):
                zero_v[r, pl.ds(j, 16)] = jnp.zeros((16,), jnp.float32)

        base_rows = s * rows_per_sub

        @pl.loop(0, rows_per_sub // 128)
        def _(b):
            pltpu.sync_copy(zero_v, acc.at[pl.ds(base_rows + b * 128, 128)])

        plsc.subcore_barrier()

        for j in range(_NBUF):
            pltpu.async_copy(p_hbm.at[sidx.at[j]], rows[j], gsem[j])

        @pl.loop(0, nb, step=_NBUF)
        def _(b):
            for j in range(_NBUF):
                blk = b + j
                pltpu.make_async_copy(p_hbm.at[sidx.at[blk]], rows[j],
                                      gsem[j]).wait()
                pltpu.sync_copy(rows[j], acc.at[didx.at[blk]], add=True)

                @pl.when(blk + _NBUF < nb)
                def _():
                    pltpu.async_copy(p_hbm.at[sidx.at[blk + _NBUF]], rows[j],
                                     gsem[j])

        plsc.subcore_barrier()
        pltpu.sync_copy(
            acc.at[pl.ds(base_rows, rows_per_sub)],
            out_hbm.at[c].at[pl.ds(base_rows, rows_per_sub)],
        )

    return edge_kernel(p, src_t, dst_t)


def _embed_body(x_ref, w_ref, b_ref, o_ref):
    h = jnp.dot(x_ref[...], w_ref[...], preferred_element_type=jnp.float32)
    o_ref[...] = jnp.maximum(h + b_ref[...], 0.0)


def _p0_body(deg_ref, h_ref, w_ref, dinv_ref, p_ref):
    deg = 1.0 + deg_ref[0, :, 0] + deg_ref[1, :, 0]
    dinv = (1.0 / jnp.sqrt(deg))[:, None]
    dinv_ref[...] = dinv
    p_ref[...] = dinv * jnp.dot(h_ref[...], w_ref[...],
                                preferred_element_type=jnp.float32)


def _combine_body(part_ref, p_ref, h_ref, dinv_ref, cb_ref, g_ref, be_ref,
                  mu_ref, var_ref, wn_ref, hn_ref, pn_ref):
    dinv = dinv_ref[...]
    agg = part_ref[0] + part_ref[1] + p_ref[...]
    t = dinv * agg + cb_ref[...]
    inv = 1.0 / jnp.sqrt(var_ref[...] + 1e-5)
    t = (t - mu_ref[...]) * inv * g_ref[...] + be_ref[...]
    hn = jnp.maximum(t, 0.0) + h_ref[...]
    hn_ref[...] = hn
    pn_ref[...] = dinv * jnp.dot(hn, wn_ref[...],
                                 preferred_element_type=jnp.float32)


def kernel(x, edge_index, W_embed, b_embed, conv_W, conv_b, bn_gamma, bn_beta,
           bn_mean, bn_var):
    N, F_in = x.shape
    E = edge_index.shape[1]
    L = conv_W.shape[0]

    acc_rows = _ceil_to(N + 1, _NS * 128)
    padded_e = _ceil_to(E, _NW * _K * _NBUF)
    nb = padded_e // (_NW * _K)
    pad = padded_e - E

    src_t = jnp.concatenate(
        [edge_index[0], jnp.zeros((pad,), edge_index.dtype)]
    ).reshape(_NW, nb, _K)
    dst_t = jnp.concatenate(
        [edge_index[1], jnp.full((pad,), N, edge_index.dtype)]
    ).reshape(_NW, nb, _K)

    grid = (N // _RB,)
    row_spec = pl.BlockSpec((_RB, _H), lambda i: (i, 0))
    vec_spec = pl.BlockSpec((1, _H), lambda i: (0, 0))
    part_spec = pl.BlockSpec((_NC, _RB, _H), lambda i: (0, i, 0))
    f32 = jnp.float32

    deg_part = _sc_degree(dst_t, acc_rows, nb)

    h = pl.pallas_call(
        _embed_body,
        grid=grid,
        in_specs=[pl.BlockSpec((_RB, F_in), lambda i: (i, 0)),
                  pl.BlockSpec((F_in, _H), lambda i: (0, 0)),
                  vec_spec],
        out_specs=row_spec,
        out_shape=jax.ShapeDtypeStruct((N, _H), f32),
    )(x, W_embed, b_embed.reshape(1, _H))

    dinv, p = pl.pallas_call(
        _p0_body,
        grid=grid,
        in_specs=[pl.BlockSpec((_NC, _RB, 16), lambda i: (0, i, 0)),
                  row_spec,
                  pl.BlockSpec((_H, _H), lambda i: (0, 0))],
        out_specs=[pl.BlockSpec((_RB, 1), lambda i: (i, 0)), row_spec],
        out_shape=[jax.ShapeDtypeStruct((N, 1), f32),
                   jax.ShapeDtypeStruct((N, _H), f32)],
    )(deg_part, h, conv_W[0])

    combine = pl.pallas_call(
        _combine_body,
        grid=grid,
        in_specs=[part_spec, row_spec, row_spec,
                  pl.BlockSpec((_RB, 1), lambda i: (i, 0)),
                  vec_spec, vec_spec, vec_spec, vec_spec, vec_spec,
                  pl.BlockSpec((_H, _H), lambda i: (0, 0))],
        out_specs=[row_spec, row_spec],
        out_shape=[jax.ShapeDtypeStruct((N, _H), f32),
                   jax.ShapeDtypeStruct((N, _H), f32)],
    )

    g2 = bn_gamma.reshape(1, _H)
    be2 = bn_beta.reshape(1, _H)
    mu2 = bn_mean.reshape(1, _H)
    var2 = bn_var.reshape(1, _H)

    for i in range(L):
        part = _sc_edge_pass(p, src_t, dst_t, acc_rows, nb)
        w_next = conv_W[(i + 1) % L]
        h, p = combine(part, p, h, dinv, conv_b[i].reshape(1, _H),
                       g2, be2, mu2, var2, w_next)
    return h


# speedup vs baseline: 38.9188x; 2.8795x over previous
import dataclasses
import functools

import jax
import jax.numpy as jnp
from jax import lax
from jax.experimental import pallas as pl
from jax.experimental.pallas import tpu as pltpu
from jax.experimental.pallas import tpu_sc as plsc

_NC = 2
_NS = 16
_NW = _NC * _NS
_K = 128
_H = 64
_RB = 1000


def _ceil_to(a, m):
    return (a + m - 1) // m * m


def _sc_mesh():
    return plsc.VectorSubcoreMesh(core_axis_name="c", subcore_axis_name="s")


def _sc_params():
    return pltpu.CompilerParams(use_tc_tiling_on_sc=False)


def _sc_degree(dst_t, acc_rows, nb):
    rows_per_sub = acc_rows // _NS
    chunk = 8

    @functools.partial(
        pl.kernel,
        out_type=jax.ShapeDtypeStruct((_NC, acc_rows, 16), jnp.float32),
        mesh=_sc_mesh(),
        compiler_params=_sc_params(),
        scratch_types=[
            pltpu.VMEM((nb, _K), jnp.int32),
            pltpu.VMEM((_K, 16), jnp.float32),
            pltpu.VMEM((128, 16), jnp.float32),
            pltpu.SemaphoreType.DMA,
            pltpu.VMEM_SHARED((acc_rows, 16), jnp.float32),
        ],
    )
    def deg_kernel(dst_hbm, out_hbm, didx, ones_v, zero_v, sem, acc):
        c = lax.axis_index("c")
        s = lax.axis_index("s")
        wid = s * _NC + c

        pltpu.sync_copy(dst_hbm.at[wid], didx)

        @pl.loop(0, _K)
        def _(r):
            ones_v[r, pl.ds(0, 16)] = jnp.full((16,), 1.0, jnp.float32)

        @pl.loop(0, 128)
        def _(r):
            zero_v[r, pl.ds(0, 16)] = jnp.zeros((16,), jnp.float32)

        base_rows = s * rows_per_sub

        @pl.loop(0, rows_per_sub // 128)
        def _(b):
            pltpu.sync_copy(zero_v, acc.at[pl.ds(base_rows + b * 128, 128)])

        plsc.subcore_barrier()

        @pl.loop(0, nb, step=chunk)
        def _(b):
            @pl.loop(0, chunk)
            def _(j):
                pltpu.async_copy(ones_v, acc.at[didx.at[b + j]], sem,
                                 add=True)

            @pl.loop(0, chunk)
            def _(j):
                pltpu.make_async_copy(ones_v, acc.at[didx.at[b + j]],
                                      sem).wait()

        plsc.subcore_barrier()
        pltpu.sync_copy(
            acc.at[pl.ds(base_rows, rows_per_sub)],
            out_hbm.at[c].at[pl.ds(base_rows, rows_per_sub)],
        )

    return deg_kernel(dst_t)


_NBUF = 4


def _sc_edge_pass(p, src_t, dst_t, acc_rows, nb):
    rows_per_sub = acc_rows // _NS

    @functools.partial(
        pl.kernel,
        out_type=jax.ShapeDtypeStruct((_NC, acc_rows, _H), jnp.float32),
        mesh=_sc_mesh(),
        compiler_params=_sc_params(),
        scratch_types=[
            pltpu.VMEM((nb, _K), jnp.int32),
            pltpu.VMEM((nb, _K), jnp.int32),
            [pltpu.VMEM((_K, _H), jnp.float32) for _ in range(_NBUF)],
            [pltpu.SemaphoreType.DMA for _ in range(_NBUF)],
            pltpu.VMEM((128, _H), jnp.float32),
            pltpu.VMEM_SHARED((acc_rows, _H), jnp.float32),
        ],
    )
    def edge_kernel(p_hbm, src_hbm, dst_hbm, out_hbm, sidx, didx, rows, gsem,
                    zero_v, acc):
        c = lax.axis_index("c")
        s = lax.axis_index("s")
        wid = s * _NC + c

        pltpu.sync_copy(src_hbm.at[wid], sidx)
        pltpu.sync_copy(dst_hbm.at[wid], didx)

        @pl.loop(0, 128)
        def _(r):
            @pl.loop(0, _H, step=16)
            def _(j):
                zero_v[r, pl.ds(j, 16)] = jnp.zeros((16,), jnp.float32)

        base_rows = s * rows_per_sub

        @pl.loop(0, rows_per_sub // 128)
        def _(b):
            pltpu.sync_copy(zero_v, acc.at[pl.ds(base_rows + b * 128, 128)])

        plsc.subcore_barrier()

        for j in range(_NBUF):
            pltpu.async_copy(p_hbm.at[sidx.at[j]], rows[j], gsem[j])

        @pl.loop(0, nb, step=_NBUF)
        def _(b):
            for j in range(_NBUF):
                blk = b + j
                pltpu.make_async_copy(p_hbm.at[sidx.at[blk]], rows[j],
                                      gsem[j]).wait()
                pltpu.sync_copy(rows[j], acc.at[didx.at[blk]], add=True)

                @pl.when(blk + _NBUF < nb)
                def _():
                    pltpu.async_copy(p_hbm.at[sidx.at[blk + _NBUF]], rows[j],
                                     gsem[j])

        plsc.subcore_barrier()
        pltpu.sync_copy(
            acc.at[pl.ds(base_rows, rows_per_sub)],
            out_hbm.at[c].at[pl.ds(base_rows, rows_per_sub)],
        )

    return edge_kernel(p, src_t, dst_t)


def _embed_body(x_ref, w_ref, b_ref, o_ref):
    h = jnp.dot(x_ref[...], w_ref[...], preferred_element_type=jnp.float32)
    o_ref[...] = jnp.maximum(h + b_ref[...], 0.0)


def _p0_body(deg_ref, h_ref, w_ref, dinv_ref, p_ref):
    deg = 1.0 + deg_ref[0, :, 0] + deg_ref[1, :, 0]
    dinv = (1.0 / jnp.sqrt(deg))[:, None]
    dinv_ref[...] = dinv
    p_ref[...] = dinv * jnp.dot(h_ref[...], w_ref[...],
                                preferred_element_type=jnp.float32)


def _combine_body(part_ref, p_ref, h_ref, dinv_ref, cb_ref, g_ref, be_ref,
                  mu_ref, var_ref, wn_ref, hn_ref, pn_ref):
    dinv = dinv_ref[...]
    agg = part_ref[0] + part_ref[1] + p_ref[...]
    t = dinv * agg + cb_ref[...]
    inv = 1.0 / jnp.sqrt(var_ref[...] + 1e-5)
    t = (t - mu_ref[...]) * inv * g_ref[...] + be_ref[...]
    hn = jnp.maximum(t, 0.0) + h_ref[...]
    hn_ref[...] = hn
    pn_ref[...] = dinv * jnp.dot(hn, wn_ref[...],
                                 preferred_element_type=jnp.float32)


def kernel(x, edge_index, W_embed, b_embed, conv_W, conv_b, bn_gamma, bn_beta,
           bn_mean, bn_var):
    N, F_in = x.shape
    E = edge_index.shape[1]
    L = conv_W.shape[0]

    acc_rows = _ceil_to(N + 1, _NS * 128)
    padded_e = _ceil_to(E, _NW * _K * _NBUF)
    nb = padded_e // (_NW * _K)
    pad = padded_e - E

    pad_iota = jnp.arange(pad, dtype=edge_index.dtype)
    src_t = jnp.concatenate(
        [edge_index[0], pad_iota % N]
    ).reshape(_NW, nb, _K)
    dst_t = jnp.concatenate(
        [edge_index[1], N + pad_iota % (acc_rows - N)]
    ).reshape(_NW, nb, _K)

    grid = (N // _RB,)
    row_spec = pl.BlockSpec((_RB, _H), lambda i: (i, 0))
    vec_spec = pl.BlockSpec((1, _H), lambda i: (0, 0))
    part_spec = pl.BlockSpec((_NC, _RB, _H), lambda i: (0, i, 0))
    f32 = jnp.float32

    deg_part = _sc_degree(dst_t, acc_rows, nb)

    h = pl.pallas_call(
        _embed_body,
        grid=grid,
        in_specs=[pl.BlockSpec((_RB, F_in), lambda i: (i, 0)),
                  pl.BlockSpec((F_in, _H), lambda i: (0, 0)),
                  vec_spec],
        out_specs=row_spec,
        out_shape=jax.ShapeDtypeStruct((N, _H), f32),
    )(x, W_embed, b_embed.reshape(1, _H))

    dinv, p = pl.pallas_call(
        _p0_body,
        grid=grid,
        in_specs=[pl.BlockSpec((_NC, _RB, 16), lambda i: (0, i, 0)),
                  row_spec,
                  pl.BlockSpec((_H, _H), lambda i: (0, 0))],
        out_specs=[pl.BlockSpec((_RB, 1), lambda i: (i, 0)), row_spec],
        out_shape=[jax.ShapeDtypeStruct((N, 1), f32),
                   jax.ShapeDtypeStruct((N, _H), f32)],
    )(deg_part, h, conv_W[0])

    combine = pl.pallas_call(
        _combine_body,
        grid=grid,
        in_specs=[part_spec, row_spec, row_spec,
                  pl.BlockSpec((_RB, 1), lambda i: (i, 0)),
                  vec_spec, vec_spec, vec_spec, vec_spec, vec_spec,
                  pl.BlockSpec((_H, _H), lambda i: (0, 0))],
        out_specs=[row_spec, row_spec],
        out_shape=[jax.ShapeDtypeStruct((N, _H), f32),
                   jax.ShapeDtypeStruct((N, _H), f32)],
    )

    g2 = bn_gamma.reshape(1, _H)
    be2 = bn_beta.reshape(1, _H)
    mu2 = bn_mean.reshape(1, _H)
    var2 = bn_var.reshape(1, _H)

    for i in range(L):
        part = _sc_edge_pass(p, src_t, dst_t, acc_rows, nb)
        w_next = conv_W[(i + 1) % L]
        h, p = combine(part, p, h, dinv, conv_b[i].reshape(1, _H),
                       g2, be2, mu2, var2, w_next)
    return h
